# baseline (speedup 1.0000x reference)
import functools

import jax
import jax.numpy as jnp
from jax import lax
from jax.experimental import pallas as pl
from jax.experimental.pallas import tpu as pltpu
from jax.experimental.pallas import tpu_sc as plsc

N = 10000
E = 320000
NC = 2
NS = 16
NW = NC * NS
E_PER_TILE = E // NW
CHUNK = 80
N_CHUNKS = E_PER_TILE // CHUNK
NBUF = 2
NIDX = 4
ZROWS = 80
N_ROWBLK = N // ZROWS


def _scatter_add_sc(h, idx, D):
    mesh = plsc.VectorSubcoreMesh(core_axis_name="c", subcore_axis_name="s")

    @functools.partial(
        pl.kernel,
        out_type=jax.ShapeDtypeStruct((NC, N, D), jnp.float32),
        mesh=mesh,
        scratch_types=[
            [pltpu.VMEM((CHUNK,), jnp.int32) for _ in range(NIDX)],
            [pltpu.VMEM((CHUNK,), jnp.int32) for _ in range(NIDX)],
            [pltpu.VMEM((CHUNK, D), jnp.float32) for _ in range(NBUF)],
            [pltpu.SemaphoreType.DMA for _ in range(NIDX)],
            [pltpu.SemaphoreType.DMA for _ in range(NIDX)],
            [pltpu.SemaphoreType.DMA for _ in range(NBUF)],
            [pltpu.SemaphoreType.DMA for _ in range(NBUF)],
            pltpu.VMEM((ZROWS, D), jnp.float32),
            pltpu.VMEM_SHARED((N, D), jnp.float32),
        ],
        compiler_params=pltpu.CompilerParams(use_tc_tiling_on_sc=False),
    )
    def k(h_hbm, idx_hbm, out_hbm, srcs, dsts, rows, s_sems, d_sems, gsems,
          ssems, zero_v, acc):
        c = lax.axis_index("c")
        s = lax.axis_index("s")
        wid = c * NS + s

        zvec = jnp.zeros((16,), jnp.float32)

        @pl.loop(0, ZROWS)
        def _(r):
            for cc in range(D // 16):
                zero_v[r, pl.ds(cc * 16, 16)] = zvec

        @pl.loop(s, N_ROWBLK, step=NS)
        def _(j):
            pltpu.sync_copy(zero_v, acc.at[pl.ds(j * ZROWS, ZROWS)])

        plsc.subcore_barrier()

        base0 = wid * E_PER_TILE

        def load_idx(j, sl):
            b = base0 + j * CHUNK
            pltpu.async_copy(idx_hbm.at[0].at[pl.ds(b, CHUNK)], srcs[sl],
                             s_sems[sl])
            pltpu.async_copy(idx_hbm.at[1].at[pl.ds(b, CHUNK)], dsts[sl],
                             d_sems[sl])

        def wait_idx(sl):
            pltpu.make_async_copy(idx_hbm.at[0].at[pl.ds(0, CHUNK)],
                                  srcs[sl], s_sems[sl]).wait()
            pltpu.make_async_copy(idx_hbm.at[1].at[pl.ds(0, CHUNK)],
                                  dsts[sl], d_sems[sl]).wait()

        def start_gather(sl, b):
            pltpu.async_copy(h_hbm.at[srcs[sl]], rows[b], gsems[b])

        def wait_gather(sl, b):
            pltpu.make_async_copy(h_hbm.at[srcs[sl]], rows[b],
                                  gsems[b]).wait()

        def start_scatter(sl, b):
            pltpu.async_copy(rows[b], acc.at[dsts[sl]], ssems[b], add=True)

        def wait_scatter(sl, b):
            pltpu.make_async_copy(rows[b], acc.at[dsts[sl]],
                                  ssems[b]).wait()

        load_idx(0, 0)
        load_idx(1, 1)
        load_idx(2, 2)
        wait_idx(0)
        start_gather(0, 0)

        @pl.loop(0, N_CHUNKS // NIDX)
        def _(g):
            j0 = g * NIDX
            for q in range(NIDX):
                j = j0 + q
                b = q % NBUF
                bn = (b + 1) % NBUF
                qn = (q + 1) % NIDX
                qp = (q + 3) % NIDX

                @pl.when(j >= 1)
                def _():
                    wait_scatter(qp, bn)

                @pl.when(j + 1 < N_CHUNKS)
                def _():
                    wait_idx(qn)
                    start_gather(qn, bn)

                wait_gather(q, b)

                @pl.when(j + 3 < N_CHUNKS)
                def _():
                    load_idx(j + 3, qp)

                start_scatter(q, b)

        _t = N_CHUNKS - 1
        wait_gather(_t % NIDX, _t % NBUF)
        start_scatter(_t % NIDX, _t % NBUF)
        wait_scatter((_t - 1) % NIDX, (_t - 1) % NBUF)
        wait_scatter(_t % NIDX, _t % NBUF)

        plsc.subcore_barrier()

        @pl.loop(s, N_ROWBLK, step=NS)
        def _(j):
            r0 = j * ZROWS
            pltpu.sync_copy(acc.at[pl.ds(r0, ZROWS)], zero_v)
            pltpu.sync_copy(zero_v, out_hbm.at[c].at[pl.ds(r0, ZROWS)])

    return k(h, idx)


def _proj_body(ix_ref, wp_ref, bp_ref, w1_ref, o_ref):
    wf = lax.dot_general(wp_ref[...], w1_ref[...], (((0,), (0,)), ((), ())),
                         preferred_element_type=jnp.float32)
    bf = lax.dot_general(bp_ref[...], w1_ref[...], (((1,), (0,)), ((), ())),
                         preferred_element_type=jnp.float32)
    o_ref[...] = lax.dot_general(ix_ref[...], wf, (((1,), (0,)), ((), ())),
                                 preferred_element_type=jnp.float32) + bf


def _layer2_body(p_ref, w2_ref, o_ref):
    x1 = jnp.maximum(p_ref[0] + p_ref[1], 0.0)
    o_ref[...] = lax.dot_general(x1, w2_ref[...], (((1,), (0,)), ((), ())),
                                 preferred_element_type=jnp.float32)


def _sum_body(q_ref, o_ref):
    o_ref[...] = q_ref[0] + q_ref[1]


def kernel(inputx, adj, nums, Wp, bp, W1, W2):
    del nums
    idx = adj
    nfeat = W1.shape[0]
    nhid = W1.shape[1]
    nclass = W2.shape[1]

    h1 = pl.pallas_call(
        _proj_body,
        out_shape=jax.ShapeDtypeStruct((N, nhid), jnp.float32),
    )(inputx, Wp, bp.reshape(1, nfeat), W1)

    p = _scatter_add_sc(h1, idx, nhid)

    blk = 1000
    h2 = pl.pallas_call(
        _layer2_body,
        grid=(N // blk,),
        in_specs=[
            pl.BlockSpec((NC, blk, nhid), lambda i: (0, i, 0)),
            pl.BlockSpec((nhid, nclass), lambda i: (0, 0)),
        ],
        out_specs=pl.BlockSpec((blk, nclass), lambda i: (i, 0)),
        out_shape=jax.ShapeDtypeStruct((N, nclass), jnp.float32),
    )(p, W2)

    q = _scatter_add_sc(h2, idx, nclass)

    out = pl.pallas_call(
        _sum_body,
        out_shape=jax.ShapeDtypeStruct((N, nclass), jnp.float32),
    )(q)
    return out


# speedup vs baseline: 1.0966x; 1.0966x over previous
import functools

import jax
import jax.numpy as jnp
from jax import lax
from jax.experimental import pallas as pl
from jax.experimental.pallas import tpu as pltpu
from jax.experimental.pallas import tpu_sc as plsc

N = 10000
E = 320000
NC = 2
NS = 16
NW = NC * NS
E_PER_TILE = E // NW
CHUNK = 128
CHUNKS_TOTAL = E // CHUNK
K_FULL = CHUNKS_TOTAL // NW
N_EXTRA = CHUNKS_TOTAL - K_FULL * NW
NBUF = 2
NIDX = 4
ZROWS = 80
N_ROWBLK = N // ZROWS


def _scatter_add_sc(h, idx, D):
    mesh = plsc.VectorSubcoreMesh(core_axis_name="c", subcore_axis_name="s")

    @functools.partial(
        pl.kernel,
        out_type=jax.ShapeDtypeStruct((NC, N, D), jnp.float32),
        mesh=mesh,
        scratch_types=[
            [pltpu.VMEM((CHUNK,), jnp.int32) for _ in range(NIDX)],
            [pltpu.VMEM((CHUNK,), jnp.int32) for _ in range(NIDX)],
            [pltpu.VMEM((CHUNK, D), jnp.float32) for _ in range(NBUF)],
            [pltpu.SemaphoreType.DMA for _ in range(NIDX)],
            [pltpu.SemaphoreType.DMA for _ in range(NIDX)],
            [pltpu.SemaphoreType.DMA for _ in range(NBUF)],
            [pltpu.SemaphoreType.DMA for _ in range(NBUF)],
            pltpu.VMEM((ZROWS, D), jnp.float32),
            pltpu.VMEM_SHARED((N, D), jnp.float32),
        ],
        compiler_params=pltpu.CompilerParams(use_tc_tiling_on_sc=False),
    )
    def k(h_hbm, idx_hbm, out_hbm, srcs, dsts, rows, s_sems, d_sems, gsems,
          ssems, zero_v, acc):
        c = lax.axis_index("c")
        s = lax.axis_index("s")
        wid = c * NS + s

        zvec = jnp.zeros((16,), jnp.float32)

        @pl.loop(0, ZROWS)
        def _(r):
            for cc in range(D // 16):
                zero_v[r, pl.ds(cc * 16, 16)] = zvec

        @pl.loop(s, N_ROWBLK, step=NS)
        def _(j):
            pltpu.sync_copy(zero_v, acc.at[pl.ds(j * ZROWS, ZROWS)])

        plsc.subcore_barrier()

        def load_chunk(cid, sl):
            o = cid * CHUNK
            pltpu.async_copy(idx_hbm.at[0].at[pl.ds(o, CHUNK)], srcs[sl],
                             s_sems[sl])
            pltpu.async_copy(idx_hbm.at[1].at[pl.ds(o, CHUNK)], dsts[sl],
                             d_sems[sl])

        def load_idx(k, sl):
            load_chunk(wid + NW * k, sl)

        def wait_idx(sl):
            pltpu.make_async_copy(idx_hbm.at[0].at[pl.ds(0, CHUNK)],
                                  srcs[sl], s_sems[sl]).wait()
            pltpu.make_async_copy(idx_hbm.at[1].at[pl.ds(0, CHUNK)],
                                  dsts[sl], d_sems[sl]).wait()

        def start_gather(sl, b):
            pltpu.async_copy(h_hbm.at[srcs[sl]], rows[b], gsems[b])

        def wait_gather(sl, b):
            pltpu.make_async_copy(h_hbm.at[srcs[sl]], rows[b],
                                  gsems[b]).wait()

        def start_scatter(sl, b):
            pltpu.async_copy(rows[b], acc.at[dsts[sl]], ssems[b], add=True)

        def wait_scatter(sl, b):
            pltpu.make_async_copy(rows[b], acc.at[dsts[sl]],
                                  ssems[b]).wait()

        def step(k, q, static):
            b = q % NBUF
            bn = (b + 1) % NBUF
            qn = (q + 1) % NIDX
            qp = (q + 3) % NIDX
            if static:
                if k >= 1:
                    wait_scatter(qp, bn)
                if k + 1 < K_FULL:
                    wait_idx(qn)
                    start_gather(qn, bn)
                wait_gather(q, b)
                if k + 3 < K_FULL:
                    load_idx(k + 3, qp)
                start_scatter(q, b)
            else:
                @pl.when(k >= 1)
                def _():
                    wait_scatter(qp, bn)

                @pl.when(k + 1 < K_FULL)
                def _():
                    wait_idx(qn)
                    start_gather(qn, bn)

                wait_gather(q, b)

                @pl.when(k + 3 < K_FULL)
                def _():
                    load_idx(k + 3, qp)

                start_scatter(q, b)

        load_idx(0, 0)
        load_idx(1, 1)
        load_idx(2, 2)
        wait_idx(0)
        start_gather(0, 0)

        n_grp = K_FULL // NIDX

        @pl.loop(0, n_grp)
        def _(g):
            j0 = g * NIDX
            for q in range(NIDX):
                step(j0 + q, q, static=False)

        for k in range(n_grp * NIDX, K_FULL):
            step(k, k % NIDX, static=True)

        wait_scatter((K_FULL - 1) % NIDX, (K_FULL - 1) % NBUF)

        @pl.when(wid < N_EXTRA)
        def _():
            load_chunk(NW * K_FULL + wid, 0)
            wait_idx(0)
            start_gather(0, 0)
            wait_gather(0, 0)
            start_scatter(0, 0)
            wait_scatter(0, 0)

        plsc.subcore_barrier()

        @pl.loop(s, N_ROWBLK, step=NS)
        def _(j):
            r0 = j * ZROWS
            pltpu.sync_copy(acc.at[pl.ds(r0, ZROWS)], zero_v)
            pltpu.sync_copy(zero_v, out_hbm.at[c].at[pl.ds(r0, ZROWS)])

    return k(h, idx)


def _proj_body(ix_ref, wp_ref, bp_ref, w1_ref, o_ref):
    wf = lax.dot_general(wp_ref[...], w1_ref[...], (((0,), (0,)), ((), ())),
                         preferred_element_type=jnp.float32)
    bf = lax.dot_general(bp_ref[...], w1_ref[...], (((1,), (0,)), ((), ())),
                         preferred_element_type=jnp.float32)
    o_ref[...] = lax.dot_general(ix_ref[...], wf, (((1,), (0,)), ((), ())),
                                 preferred_element_type=jnp.float32) + bf


def _layer2_body(p_ref, w2_ref, o_ref):
    x1 = jnp.maximum(p_ref[0] + p_ref[1], 0.0)
    o_ref[...] = lax.dot_general(x1, w2_ref[...], (((1,), (0,)), ((), ())),
                                 preferred_element_type=jnp.float32)


def _sum_body(q_ref, o_ref):
    o_ref[...] = q_ref[0] + q_ref[1]


def kernel(inputx, adj, nums, Wp, bp, W1, W2):
    del nums
    idx = adj
    nfeat = W1.shape[0]
    nhid = W1.shape[1]
    nclass = W2.shape[1]

    h1 = pl.pallas_call(
        _proj_body,
        out_shape=jax.ShapeDtypeStruct((N, nhid), jnp.float32),
    )(inputx, Wp, bp.reshape(1, nfeat), W1)

    p = _scatter_add_sc(h1, idx, nhid)

    blk = 1000
    h2 = pl.pallas_call(
        _layer2_body,
        grid=(N // blk,),
        in_specs=[
            pl.BlockSpec((NC, blk, nhid), lambda i: (0, i, 0)),
            pl.BlockSpec((nhid, nclass), lambda i: (0, 0)),
        ],
        out_specs=pl.BlockSpec((blk, nclass), lambda i: (i, 0)),
        out_shape=jax.ShapeDtypeStruct((N, nclass), jnp.float32),
    )(p, W2)

    q = _scatter_add_sc(h2, idx, nclass)

    out = pl.pallas_call(
        _sum_body,
        out_shape=jax.ShapeDtypeStruct((N, nclass), jnp.float32),
    )(q)
    return out


# speedup vs baseline: 1.4298x; 1.3038x over previous
import functools

import jax
import jax.numpy as jnp
from jax import lax
from jax.experimental import pallas as pl
from jax.experimental.pallas import tpu as pltpu
from jax.experimental.pallas import tpu_sc as plsc

N = 10000
E = 320000
NC = 2
NS = 16
NW = NC * NS
RAW = 6
DPAD = 16
CHUNK = 128
CHUNKS_TOTAL = E // CHUNK
K_FULL = CHUNKS_TOTAL // NW
N_EXTRA = CHUNKS_TOTAL - K_FULL * NW
NBUF = 2
NIDX = 4
ZROWS = 80
N_ROWBLK = N // ZROWS


def _scatter_add_sc(h, idx, D):
    mesh = plsc.VectorSubcoreMesh(core_axis_name="c", subcore_axis_name="s")

    @functools.partial(
        pl.kernel,
        out_type=jax.ShapeDtypeStruct((NC, N, D), jnp.float32),
        mesh=mesh,
        scratch_types=[
            [pltpu.VMEM((CHUNK,), jnp.int32) for _ in range(NIDX)],
            [pltpu.VMEM((CHUNK,), jnp.int32) for _ in range(NIDX)],
            [pltpu.VMEM((CHUNK, D), jnp.float32) for _ in range(NBUF)],
            [pltpu.SemaphoreType.DMA for _ in range(NIDX)],
            [pltpu.SemaphoreType.DMA for _ in range(NIDX)],
            [pltpu.SemaphoreType.DMA for _ in range(NBUF)],
            [pltpu.SemaphoreType.DMA for _ in range(NBUF)],
            pltpu.VMEM((ZROWS, D), jnp.float32),
            pltpu.VMEM_SHARED((N, D), jnp.float32),
        ],
        compiler_params=pltpu.CompilerParams(use_tc_tiling_on_sc=False),
    )
    def k(h_hbm, idx_hbm, out_hbm, srcs, dsts, rows, s_sems, d_sems, gsems,
          ssems, zero_v, acc):
        c = lax.axis_index("c")
        s = lax.axis_index("s")
        wid = c * NS + s

        zvec = jnp.zeros((16,), jnp.float32)

        @pl.loop(0, ZROWS)
        def _(r):
            for cc in range(D // 16):
                zero_v[r, pl.ds(cc * 16, 16)] = zvec

        @pl.loop(s, N_ROWBLK, step=NS)
        def _(j):
            pltpu.sync_copy(zero_v, acc.at[pl.ds(j * ZROWS, ZROWS)])

        plsc.subcore_barrier()

        def load_chunk(cid, sl):
            o = cid * CHUNK
            pltpu.async_copy(idx_hbm.at[0].at[pl.ds(o, CHUNK)], srcs[sl],
                             s_sems[sl])
            pltpu.async_copy(idx_hbm.at[1].at[pl.ds(o, CHUNK)], dsts[sl],
                             d_sems[sl])

        def load_idx(k, sl):
            load_chunk(wid + NW * k, sl)

        def wait_idx(sl):
            pltpu.make_async_copy(idx_hbm.at[0].at[pl.ds(0, CHUNK)],
                                  srcs[sl], s_sems[sl]).wait()
            pltpu.make_async_copy(idx_hbm.at[1].at[pl.ds(0, CHUNK)],
                                  dsts[sl], d_sems[sl]).wait()

        def start_gather(sl, b):
            pltpu.async_copy(h_hbm.at[srcs[sl]], rows[b], gsems[b])

        def wait_gather(sl, b):
            pltpu.make_async_copy(h_hbm.at[srcs[sl]], rows[b],
                                  gsems[b]).wait()

        def start_scatter(sl, b):
            pltpu.async_copy(rows[b], acc.at[dsts[sl]], ssems[b], add=True)

        def wait_scatter(sl, b):
            pltpu.make_async_copy(rows[b], acc.at[dsts[sl]],
                                  ssems[b]).wait()

        def step(k, q, static):
            b = q % NBUF
            bn = (b + 1) % NBUF
            qn = (q + 1) % NIDX
            qp = (q + 3) % NIDX
            if static:
                if k >= 1:
                    wait_scatter(qp, bn)
                if k + 1 < K_FULL:
                    wait_idx(qn)
                    start_gather(qn, bn)
                wait_gather(q, b)
                if k + 3 < K_FULL:
                    load_idx(k + 3, qp)
                start_scatter(q, b)
            else:
                @pl.when(k >= 1)
                def _():
                    wait_scatter(qp, bn)

                @pl.when(k + 1 < K_FULL)
                def _():
                    wait_idx(qn)
                    start_gather(qn, bn)

                wait_gather(q, b)

                @pl.when(k + 3 < K_FULL)
                def _():
                    load_idx(k + 3, qp)

                start_scatter(q, b)

        load_idx(0, 0)
        load_idx(1, 1)
        load_idx(2, 2)
        wait_idx(0)
        start_gather(0, 0)

        n_grp = K_FULL // NIDX

        @pl.loop(0, n_grp)
        def _(g):
            j0 = g * NIDX
            for q in range(NIDX):
                step(j0 + q, q, static=False)

        for k in range(n_grp * NIDX, K_FULL):
            step(k, k % NIDX, static=True)

        wait_scatter((K_FULL - 1) % NIDX, (K_FULL - 1) % NBUF)

        @pl.when(wid < N_EXTRA)
        def _():
            load_chunk(NW * K_FULL + wid, 0)
            wait_idx(0)
            start_gather(0, 0)
            wait_gather(0, 0)
            start_scatter(0, 0)
            wait_scatter(0, 0)

        plsc.subcore_barrier()

        @pl.loop(s, N_ROWBLK, step=NS)
        def _(j):
            r0 = j * ZROWS
            pltpu.sync_copy(acc.at[pl.ds(r0, ZROWS)], zero_v)
            pltpu.sync_copy(zero_v, out_hbm.at[c].at[pl.ds(r0, ZROWS)])

    return k(h, idx)


def _fused_body(p_ref, wp_ref, bp_ref, w1_ref, w2_ref, o_ref):
    wf = lax.dot_general(wp_ref[...], w1_ref[...], (((0,), (0,)), ((), ())),
                         preferred_element_type=jnp.float32)
    bf = lax.dot_general(bp_ref[...], w1_ref[...], (((1,), (0,)), ((), ())),
                         preferred_element_type=jnp.float32)
    ps = p_ref[0] + p_ref[1]
    agg = lax.dot_general(ps[:, :RAW], wf, (((1,), (0,)), ((), ())),
                          preferred_element_type=jnp.float32)
    agg = agg + ps[:, RAW:RAW + 1] * bf
    x1 = jnp.maximum(agg, 0.0)
    o_ref[...] = lax.dot_general(x1, w2_ref[...], (((1,), (0,)), ((), ())),
                                 preferred_element_type=jnp.float32)


def _sum_body(q_ref, o_ref):
    o_ref[...] = q_ref[0] + q_ref[1]


def kernel(inputx, adj, nums, Wp, bp, W1, W2):
    del nums
    idx = adj
    nfeat = W1.shape[0]
    nhid = W1.shape[1]
    nclass = W2.shape[1]

    ix16 = jnp.concatenate(
        [inputx, jnp.ones((N, 1), jnp.float32),
         jnp.zeros((N, DPAD - RAW - 1), jnp.float32)], axis=1)

    p = _scatter_add_sc(ix16, idx, DPAD)

    blk = 1000
    h2 = pl.pallas_call(
        _fused_body,
        grid=(N // blk,),
        in_specs=[
            pl.BlockSpec((NC, blk, DPAD), lambda i: (0, i, 0)),
            pl.BlockSpec((nfeat, RAW), lambda i: (0, 0)),
            pl.BlockSpec((1, nfeat), lambda i: (0, 0)),
            pl.BlockSpec((nfeat, nhid), lambda i: (0, 0)),
            pl.BlockSpec((nhid, nclass), lambda i: (0, 0)),
        ],
        out_specs=pl.BlockSpec((blk, nclass), lambda i: (i, 0)),
        out_shape=jax.ShapeDtypeStruct((N, nclass), jnp.float32),
    )(p, Wp, bp.reshape(1, nfeat), W1, W2)

    q = _scatter_add_sc(h2, idx, nclass)

    out = pl.pallas_call(
        _sum_body,
        out_shape=jax.ShapeDtypeStruct((N, nclass), jnp.float32),
    )(q)
    return out
